# initial kernel scaffold (unmeasured)
import jax
import jax.numpy as jnp
from jax import lax
from jax.experimental import pallas as pl
from jax.experimental.pallas import tpu as pltpu

N_DEV = 8


def kernel(table, idx):
    rows_per, d = table.shape
    n = idx.shape[0]

    my = lax.axis_index("i")
    local = idx - my * rows_per
    partial = jnp.take(
        table.astype(jnp.bfloat16), local, axis=0, mode="fill", fill_value=0
    )

    def body(p_ref, out_ref, comm_ref, send_sems, recv_sems):
        my_pos = lax.axis_index("i")
        left = (my_pos - 1) % N_DEV
        right = (my_pos + 1) % N_DEV

        barrier_sem = pltpu.get_barrier_semaphore()
        for nbr in (left, right):
            pl.semaphore_signal(
                barrier_sem, inc=1,
                device_id=(nbr,), device_id_type=pl.DeviceIdType.MESH,
            )
        pl.semaphore_wait(barrier_sem, 2)

        out_ref[...] = p_ref[...]
        comm_ref[0] = p_ref[...]

        for h in range(N_DEV - 1):
            send_slot = h % 2
            recv_slot = (h + 1) % 2
            rdma = pltpu.make_async_remote_copy(
                src_ref=comm_ref.at[send_slot],
                dst_ref=comm_ref.at[recv_slot],
                send_sem=send_sems.at[send_slot],
                recv_sem=recv_sems.at[recv_slot],
                device_id=(right,),
                device_id_type=pl.DeviceIdType.MESH,
            )
            rdma.start()
            rdma.wait()
            out_ref[...] += comm_ref[recv_slot]

    return pl.pallas_call(
        body,
        out_shape=jax.ShapeDtypeStruct((n, d), jnp.bfloat16),
        in_specs=[pl.BlockSpec(memory_space=pltpu.VMEM)],
        out_specs=pl.BlockSpec(memory_space=pltpu.VMEM),
        scratch_shapes=[
            pltpu.VMEM((2, n, d), jnp.bfloat16),
            pltpu.SemaphoreType.DMA((2,)),
            pltpu.SemaphoreType.DMA((2,)),
        ],
        compiler_params=pltpu.CompilerParams(collective_id=0),
    )(partial)


# baseline (device time: 357687 ns/iter reference)
import jax
import jax.numpy as jnp
from jax import lax
from jax.experimental import pallas as pl
from jax.experimental.pallas import tpu as pltpu

N_DEV = 8


def kernel(table, idx):
    rows_per, d = table.shape
    n = idx.shape[0]

    my = lax.axis_index("i")
    local = idx - my * rows_per
    valid = (local >= 0) & (local < rows_per)
    safe = jnp.where(valid, local, 0)
    rows = jnp.take(table.astype(jnp.bfloat16), safe, axis=0, mode="clip")
    partial = jnp.where(valid[:, None], rows, jnp.bfloat16(0))

    def body(p_ref, out_ref, comm_ref, send_sems, recv_sems):
        my_pos = lax.axis_index("i")
        left = (my_pos - 1) % N_DEV
        right = (my_pos + 1) % N_DEV

        barrier_sem = pltpu.get_barrier_semaphore()
        for nbr in (left, right):
            pl.semaphore_signal(
                barrier_sem, inc=1,
                device_id=(nbr,), device_id_type=pl.DeviceIdType.MESH,
            )
        pl.semaphore_wait(barrier_sem, 2)

        out_ref[...] = p_ref[...]
        comm_ref[0] = p_ref[...]

        for h in range(N_DEV - 1):
            send_slot = h % 2
            recv_slot = (h + 1) % 2
            rdma = pltpu.make_async_remote_copy(
                src_ref=comm_ref.at[send_slot],
                dst_ref=comm_ref.at[recv_slot],
                send_sem=send_sems.at[send_slot],
                recv_sem=recv_sems.at[recv_slot],
                device_id=(right,),
                device_id_type=pl.DeviceIdType.MESH,
            )
            rdma.start()
            rdma.wait()
            out_ref[...] += comm_ref[recv_slot]

    return pl.pallas_call(
        body,
        out_shape=jax.ShapeDtypeStruct((n, d), jnp.bfloat16),
        in_specs=[pl.BlockSpec(memory_space=pltpu.VMEM)],
        out_specs=pl.BlockSpec(memory_space=pltpu.VMEM),
        scratch_shapes=[
            pltpu.VMEM((2, n, d), jnp.bfloat16),
            pltpu.SemaphoreType.DMA((2,)),
            pltpu.SemaphoreType.DMA((2,)),
        ],
        compiler_params=pltpu.CompilerParams(collective_id=0),
    )(partial)


# device time: 114091 ns/iter; 3.1351x vs baseline; 3.1351x over previous
import jax
import jax.numpy as jnp
from jax import lax
from jax.experimental import pallas as pl
from jax.experimental.pallas import tpu as pltpu

N_DEV = 8
_STAGES = ((3, 1), (1, 0), (4, 2))


def kernel(table, idx):
    rows_per, d = table.shape
    n = idx.shape[0]

    my = lax.axis_index("i")
    local = idx - my * rows_per
    valid = (local >= 0) & (local < rows_per)
    safe = jnp.where(valid, local, 0)
    rows = jnp.take(table.astype(jnp.bfloat16), safe, axis=0, mode="clip")
    partial = jnp.where(valid[:, None], rows, jnp.bfloat16(0))

    sizes = (n // 2, n // 4, n // 8)
    st_offs = (0, n // 2, n // 2 + n // 4)

    def body(p_ref, out_ref, stage_ref, send_sems, recv_sems):
        p = lax.axis_index("i")
        bits = [(p >> bit) & 1 for _, bit in _STAGES]

        barrier_sem = pltpu.get_barrier_semaphore()
        for m, _ in _STAGES:
            pl.semaphore_signal(
                barrier_sem, inc=1,
                device_id=(p ^ m,), device_id_type=pl.DeviceIdType.MESH,
            )
        pl.semaphore_wait(barrier_sem, 3)

        def cond_for(combo):
            c = bits[0] == combo[0]
            for j in range(1, len(combo)):
                c = jnp.logical_and(c, bits[j] == combo[j])
            return c

        def combos(k):
            out = [()]
            for _ in range(k):
                out = [c + (v,) for c in out for v in (0, 1)]
            return out

        def rs_rdma(s, combo):
            keep_base = sum(combo[j] * sizes[j] for j in range(s))
            send_off = keep_base + (1 - combo[s]) * sizes[s]
            src = p_ref if s == 0 else out_ref
            return pltpu.make_async_remote_copy(
                src_ref=src.at[pl.ds(send_off, sizes[s]), :],
                dst_ref=stage_ref.at[pl.ds(st_offs[s], sizes[s]), :],
                send_sem=send_sems.at[s],
                recv_sem=recv_sems.at[s],
                device_id=(p ^ _STAGES[s][0],),
                device_id_type=pl.DeviceIdType.MESH,
            )

        for combo in combos(1):
            @pl.when(cond_for(combo))
            def _(combo=combo):
                rs_rdma(0, combo).start()

        out_ref[...] = p_ref[...]

        for combo in combos(1):
            @pl.when(cond_for(combo))
            def _(combo=combo):
                rdma = rs_rdma(0, combo)
                rdma.wait()
                keep = combo[0] * sizes[0]
                out_ref[pl.ds(keep, sizes[0]), :] = (
                    out_ref[pl.ds(keep, sizes[0]), :]
                    + stage_ref[pl.ds(st_offs[0], sizes[0]), :]
                )

        for s in (1, 2):
            for combo in combos(s + 1):
                @pl.when(cond_for(combo))
                def _(s=s, combo=combo):
                    rdma = rs_rdma(s, combo)
                    rdma.start()
                    rdma.wait()
                    keep = sum(combo[j] * sizes[j] for j in range(s + 1))
                    out_ref[pl.ds(keep, sizes[s]), :] = (
                        out_ref[pl.ds(keep, sizes[s]), :]
                        + stage_ref[pl.ds(st_offs[s], sizes[s]), :]
                    )

        for ag_i, s in enumerate((2, 1, 0)):
            mask = _STAGES[s][0]
            for combo in combos(s + 1):
                @pl.when(cond_for(combo))
                def _(s=s, combo=combo, mask=mask, ag_i=ag_i):
                    off = sum(combo[j] * sizes[j] for j in range(s + 1))
                    rdma = pltpu.make_async_remote_copy(
                        src_ref=out_ref.at[pl.ds(off, sizes[s]), :],
                        dst_ref=out_ref.at[pl.ds(off, sizes[s]), :],
                        send_sem=send_sems.at[3 + ag_i],
                        recv_sem=recv_sems.at[3 + ag_i],
                        device_id=(p ^ mask,),
                        device_id_type=pl.DeviceIdType.MESH,
                    )
                    rdma.start()
                    rdma.wait()

    return pl.pallas_call(
        body,
        out_shape=jax.ShapeDtypeStruct((n, d), jnp.bfloat16),
        in_specs=[pl.BlockSpec(memory_space=pltpu.VMEM)],
        out_specs=pl.BlockSpec(memory_space=pltpu.VMEM),
        scratch_shapes=[
            pltpu.VMEM((n // 2 + n // 4 + n // 8, d), jnp.bfloat16),
            pltpu.SemaphoreType.DMA((6,)),
            pltpu.SemaphoreType.DMA((6,)),
        ],
        compiler_params=pltpu.CompilerParams(collective_id=0),
    )(partial)
